# Initial kernel scaffold; baseline (speedup 1.0000x reference)
import jax
import jax.numpy as jnp
from jax.experimental import pallas as pl
from jax.experimental.pallas import tpu as pltpu
from jax.experimental.pallas import tpu_sc as plsc

B = 16384
L = 50
DIM = 32
NUM_INDICES = B * L
WINDOW = 128


def kernel(input, emb_weight):
    idx = input.reshape(1, NUM_INDICES).astype(jnp.int32)

    @pl.kernel(
        out_type=jax.ShapeDtypeStruct((NUM_INDICES, DIM), emb_weight.dtype),
        mesh=plsc.VectorSubcoreMesh(
            core_axis_name="core", subcore_axis_name="subcore"
        ),
    )
    def gather_kernel(x_hbm, i_hbm, o_hbm):
        def body(i_vmem, o_vmem):
            pltpu.sync_copy(x_hbm.at[i_vmem.at[0]], o_vmem)

        pltpu.emit_pipeline(
            body,
            grid=(NUM_INDICES // WINDOW,),
            in_specs=[pl.BlockSpec((1, WINDOW), index_map=lambda i: (0, i))],
            out_specs=[pl.BlockSpec((WINDOW, DIM), index_map=lambda i: (i, 0))],
            core_axis_name=("core", "subcore"),
            dimension_semantics=(pltpu.PARALLEL,),
        )(i_hbm, o_hbm)

    out = gather_kernel(emb_weight, idx)
    return out.reshape(B, L, DIM)


# speedup vs baseline: 1.1109x; 1.1109x over previous
import functools

import jax
import jax.numpy as jnp
from jax import lax
from jax.experimental import pallas as pl
from jax.experimental.pallas import tpu as pltpu
from jax.experimental.pallas import tpu_sc as plsc

B = 16384
L = 50
DIM = 32
N = B * L
NC = 2
NS = 16
NW = NC * NS
PER_W = N // NW
CHUNK = 3200
STEPS = PER_W // CHUNK


def kernel(input, emb_weight):
    idx = input.reshape(N).astype(jnp.int32)

    @functools.partial(
        pl.kernel,
        out_type=jax.ShapeDtypeStruct((N, DIM), jnp.float32),
        mesh=plsc.VectorSubcoreMesh(core_axis_name="c", subcore_axis_name="s"),
        compiler_params=pltpu.CompilerParams(use_tc_tiling_on_sc=False),
        scratch_types=[
            pltpu.VMEM((CHUNK,), jnp.int32),
            pltpu.VMEM((CHUNK, DIM), jnp.float32),
            pltpu.SemaphoreType.DMA,
        ],
    )
    def gather_kernel(table_hbm, idx_hbm, out_hbm, idx_v, rows_v, sem):
        wid = lax.axis_index("s") * NC + lax.axis_index("c")
        base = wid * PER_W

        @pl.loop(0, STEPS)
        def _(c):
            off = base + c * CHUNK
            pltpu.sync_copy(idx_hbm.at[pl.ds(off, CHUNK)], idx_v)
            pltpu.async_copy(table_hbm.at[idx_v], rows_v, sem).wait()
            pltpu.sync_copy(rows_v, out_hbm.at[pl.ds(off, CHUNK)])

    out = gather_kernel(emb_weight, idx)
    return out.reshape(B, L, DIM)


# speedup vs baseline: 1.1133x; 1.0021x over previous
import functools

import jax
import jax.numpy as jnp
from jax import lax
from jax.experimental import pallas as pl
from jax.experimental.pallas import tpu as pltpu
from jax.experimental.pallas import tpu_sc as plsc

B = 16384
L = 50
DIM = 32
N = B * L
NC = 2
NS = 16
NW = NC * NS
PER_W = N // NW
CHUNK = 1280
STEPS = PER_W // CHUNK
NBUF = 3


def kernel(input, emb_weight):
    idx = input.reshape(N).astype(jnp.int32)

    @functools.partial(
        pl.kernel,
        out_type=jax.ShapeDtypeStruct((N, DIM), jnp.float32),
        mesh=plsc.VectorSubcoreMesh(core_axis_name="c", subcore_axis_name="s"),
        compiler_params=pltpu.CompilerParams(use_tc_tiling_on_sc=False),
        scratch_types=(
            [pltpu.VMEM((CHUNK,), jnp.int32) for _ in range(NBUF)]
            + [pltpu.VMEM((CHUNK, DIM), jnp.float32) for _ in range(NBUF)]
            + [pltpu.SemaphoreType.DMA for _ in range(4)]
        ),
    )
    def gather_kernel(table_hbm, idx_hbm, out_hbm,
                      i0, i1, i2, r0, r1, r2, g0, g1, w0, w1):
        idx_bufs = [i0, i1, i2]
        rows_bufs = [r0, r1, r2]
        gsems = [g0, g1]
        wsems = [w0, w1]

        wid = lax.axis_index("s") * NC + lax.axis_index("c")
        base = wid * PER_W

        def off(c):
            return base + c * CHUNK

        gathers = {}
        writes = {}

        def start_gather(c):
            b = c % NBUF
            pltpu.sync_copy(idx_hbm.at[pl.ds(off(c), CHUNK)], idx_bufs[b])
            gathers[c] = pltpu.async_copy(
                table_hbm.at[idx_bufs[b]], rows_bufs[b], gsems[c % 2]
            )

        start_gather(0)
        start_gather(1)
        for c in range(STEPS):
            b = c % NBUF
            gathers[c].wait()
            writes[c] = pltpu.async_copy(
                rows_bufs[b], out_hbm.at[pl.ds(off(c), CHUNK)], wsems[c % 2]
            )
            if c + 2 < STEPS:
                if c >= 1:
                    writes[c - 1].wait()
                start_gather(c + 2)
        writes[STEPS - 2].wait()
        writes[STEPS - 1].wait()

    out = gather_kernel(emb_weight, idx)
    return out.reshape(B, L, DIM)


# speedup vs baseline: 1.8065x; 1.6226x over previous
import functools

import jax
import jax.numpy as jnp
from jax import lax
from jax.experimental import pallas as pl
from jax.experimental.pallas import tpu as pltpu
from jax.experimental.pallas import tpu_sc as plsc

B = 16384
L = 50
DIM = 32
N = B * L
NC = 2
NS = 16
NW = NC * NS
B_PER_W = B // NW
CB = 16
CHUNK = CB * L
STEPS = B_PER_W // CB
NBUF = 3


def kernel(input, emb_weight):
    idx = input.reshape(N).astype(jnp.int32)

    @functools.partial(
        pl.kernel,
        out_type=jax.ShapeDtypeStruct((B, L, DIM), jnp.float32),
        mesh=plsc.VectorSubcoreMesh(core_axis_name="c", subcore_axis_name="s"),
        compiler_params=pltpu.CompilerParams(use_tc_tiling_on_sc=False),
        scratch_types=(
            [pltpu.VMEM((CHUNK,), jnp.int32) for _ in range(NBUF)]
            + [pltpu.VMEM((CHUNK, DIM), jnp.float32) for _ in range(NBUF)]
            + [pltpu.SemaphoreType.DMA for _ in range(4)]
        ),
    )
    def gather_kernel(table_hbm, idx_hbm, out_hbm,
                      i0, i1, i2, r0, r1, r2, g0, g1, w0, w1):
        idx_bufs = [i0, i1, i2]
        rows_bufs = [r0, r1, r2]
        gsems = [g0, g1]
        wsems = [w0, w1]

        wid = lax.axis_index("s") * NC + lax.axis_index("c")
        base_b = wid * B_PER_W

        gathers = {}

        def start_gather(c):
            b = c % NBUF
            off = (base_b + c * CB) * L
            pltpu.sync_copy(idx_hbm.at[pl.ds(off, CHUNK)], idx_bufs[b])
            gathers[c] = pltpu.async_copy(
                table_hbm.at[idx_bufs[b]], rows_bufs[b], gsems[c % 2]
            )

        def fire_writes(c):
            b = c % NBUF
            b0 = base_b + c * CB

            @pl.loop(0, CB)
            def _(j):
                pltpu.async_copy(
                    rows_bufs[b].at[pl.ds(j * L, L)],
                    out_hbm.at[b0 + j],
                    wsems[c % 2],
                )

        def drain_writes(c):
            @pl.loop(0, CB)
            def _(j):
                pltpu.make_async_copy(
                    rows_bufs[c % NBUF].at[pl.ds(0, L)],
                    out_hbm.at[base_b],
                    wsems[c % 2],
                ).wait()

        start_gather(0)
        start_gather(1)
        for c in range(STEPS):
            gathers[c].wait()
            fire_writes(c)
            if c + 2 < STEPS:
                if c >= 1:
                    drain_writes(c - 1)
                start_gather(c + 2)
        drain_writes(STEPS - 2)
        drain_writes(STEPS - 1)

    out = gather_kernel(emb_weight, idx)
    return out
